# baseline (speedup 1.0000x reference)
import functools

import jax
import jax.numpy as jnp
from jax import lax
from jax.experimental import pallas as pl
from jax.experimental.pallas import tpu as pltpu
from jax.experimental.pallas import tpu_sc as plsc

NC = 2
NS = 16
NW = NC * NS
BCH = 8


@functools.partial(jax.jit, static_argnums=(2, 3, 4))
def _sc_gather(idx_pad, table, b, l, d):
    lpad = -(-l // 8) * 8
    dpad = -(-d // 128) * 128
    lstride = idx_pad.shape[1]
    per_w_b = b // NW
    nchunk = per_w_b // BCH
    npair = nchunk // 2
    assert nchunk % 2 == 0 and nchunk >= 4
    mesh = plsc.VectorSubcoreMesh(core_axis_name="c", subcore_axis_name="s")

    @functools.partial(
        pl.kernel,
        out_type=jax.ShapeDtypeStruct((b, lpad, dpad), jnp.float32),
        mesh=mesh,
        scratch_types=[
            pltpu.VMEM((BCH, lstride), jnp.int32),
            pltpu.VMEM((BCH, lstride), jnp.int32),
            pltpu.VMEM((BCH * lpad, d), jnp.float32),
            pltpu.VMEM((BCH * lpad, d), jnp.float32),
            pltpu.SemaphoreType.DMA,
            pltpu.SemaphoreType.DMA,
            pltpu.SemaphoreType.DMA,
        ],
        compiler_params=pltpu.CompilerParams(use_tc_tiling_on_sc=False),
    )
    def k(table_hbm, idx_hbm, out_hbm, idx0, idx1, rows0, rows1, g0, g1, osem):
        wid = lax.axis_index("s") * NC + lax.axis_index("c")
        bbase = wid * per_w_b

        def idx_in(c, dst):
            pltpu.sync_copy(idx_hbm.at[pl.ds(bbase + c * BCH, BCH), :], dst)

        def gathers(idx_v, rows_v, sem):
            return [
                pltpu.make_async_copy(
                    table_hbm.at[idx_v.at[j, pl.ds(0, lpad)]],
                    rows_v.at[pl.ds(j * lpad, lpad), :],
                    sem,
                )
                for j in range(BCH)
            ]

        def fire(idx_v, rows_v, sem):
            for cp in gathers(idx_v, rows_v, sem):
                cp.start()

        def drain(idx_v, rows_v, sem):
            for cp in gathers(idx_v, rows_v, sem):
                cp.wait()

        def out_wr(c, src):
            b0 = bbase + c * BCH
            cps = [
                pltpu.make_async_copy(
                    src.at[pl.ds(j * lpad, l), :],
                    out_hbm.at[b0 + j, pl.ds(0, l), pl.ds(0, d)],
                    osem,
                )
                for j in range(BCH)
            ]
            for cp in cps:
                cp.start()
            for cp in cps:
                cp.wait()

        idx_in(0, idx0)
        fire(idx0, rows0, g0)

        @pl.loop(0, npair - 1)
        def _body(p):
            c = 2 * p
            idx_in(c + 1, idx1)
            drain(idx0, rows0, g0)
            fire(idx1, rows1, g1)
            out_wr(c, rows0)
            idx_in(c + 2, idx0)
            drain(idx1, rows1, g1)
            fire(idx0, rows0, g0)
            out_wr(c + 1, rows1)

        c = nchunk - 2
        idx_in(c + 1, idx1)
        drain(idx0, rows0, g0)
        fire(idx1, rows1, g1)
        out_wr(c, rows0)
        drain(idx1, rows1, g1)
        out_wr(c + 1, rows1)

    return k(table, idx_pad)


def kernel(indices, table):
    b, l = indices.shape
    d = table.shape[1]
    lstride = -(-l // 128) * 128
    idx_pad = jnp.pad(indices.astype(jnp.int32), ((0, 0), (0, lstride - l)))
    out = _sc_gather(idx_pad, table, b, l, d)
    return out[:, :l, :d]
